# baseline (speedup 1.0000x reference)
import jax
import jax.numpy as jnp
import numpy as np
from jax import lax
from jax.experimental import pallas as pl
from jax.experimental.pallas import tpu as pltpu
from jax.experimental.pallas import tpu_sc as plsc

N_ATOMS = 50000
N_SPLIT = 24992
NUM_TYPES = 64
_IRREPS = [(128, 1), (64, 3), (32, 5)]
NUM_SCALAR = 128
DIM = sum(m * d for m, d in _IRREPS)
NUM_IRREPS = sum(m for m, _ in _IRREPS)

_scale_idx = []
_k = 0
for _mul, _irdim in _IRREPS:
    for _ in range(_mul):
        _scale_idx += [_k] * _irdim
        _k += 1
SCALE_IDX_NP = np.asarray(_scale_idx, dtype=np.int32)
assert SCALE_IDX_NP.shape[0] == DIM

NC, NS, L = 2, 16, 16
NW = NC * NS
T = 16

_COLS30 = DIM // L


def _make_body(ntiles):
  def _body(nf_hbm, at_hbm, scales_hbm, shifts_hbm, sidx_hbm, out_hbm,
            scales_v, shifts_v, sidx_v, table_v,
            nf0, nf1, out0, out1, sp0, sp1,
            isem0, isem1, osem0, osem1):
    wid = lax.axis_index("s") * NC + lax.axis_index("c")

    pltpu.sync_copy(scales_hbm, scales_v)
    pltpu.sync_copy(shifts_hbm, shifts_v)
    pltpu.sync_copy(sidx_hbm, sidx_v)

    iota = lax.iota(jnp.int32, L)

    def expand_row(r, carry):
        rbase = jnp.full((L,), r * NUM_IRREPS, dtype=jnp.int32)
        for cb in range(_COLS30):
            col = sidx_v[pl.ds(cb * L, L)]
            vals = plsc.load_gather(scales_v, [rbase + col])
            table_v[pl.ds(r * DIM + cb * L, L)] = vals
        return carry

    lax.fori_loop(0, NUM_TYPES, expand_row, 0)

    ntw = (ntiles - 1 - wid) // NW + 1

    def tbase(t):
        return (wid + t * NW) * T

    def issue_in(t, nf_v, sp_v, isem):
        b = tbase(t)
        pltpu.async_copy(nf_hbm.at[pl.ds(b, T), :], nf_v, isem)
        pltpu.async_copy(at_hbm.at[pl.ds(b, T)], sp_v, isem)

    def compute(nf_v, sp_v, out_v):
        @plsc.parallel_loop(0, T, 1)
        def per_atom(a):
            av = jnp.full((L,), a, dtype=jnp.int32)
            sa = plsc.load_gather(sp_v, [av])
            sbase = sa * DIM + iota
            hbase = sa * NUM_SCALAR + iota
            for cb in range(NUM_SCALAR // L):
                nfc = nf_v[a, pl.ds(cb * L, L)]
                scc = plsc.load_gather(table_v, [sbase + (cb * L)])
                shc = plsc.load_gather(shifts_v, [hbase + (cb * L)])
                out_v[a, pl.ds(cb * L, L)] = nfc * scc + shc
            for cb in range(NUM_SCALAR // L, _COLS30):
                nfc = nf_v[a, pl.ds(cb * L, L)]
                scc = plsc.load_gather(table_v, [sbase + (cb * L)])
                out_v[a, pl.ds(cb * L, L)] = nfc * scc

    def slot(t, h, nf_v, sp_v, out_v, isem, osem):
        b = tbase(t)
        pltpu.make_async_copy(nf_hbm.at[pl.ds(b, T), :], nf_v, isem).wait()
        pltpu.make_async_copy(at_hbm.at[pl.ds(b, T)], sp_v, isem).wait()

        @pl.when(h > 0)
        def _():
            pltpu.make_async_copy(out_v, out_hbm.at[pl.ds(b, T), :],
                                  osem).wait()

        compute(nf_v, sp_v, out_v)
        pltpu.async_copy(out_v, out_hbm.at[pl.ds(b, T), :], osem)

        @pl.when(t + 2 < ntw)
        def _():
            issue_in(t + 2, nf_v, sp_v, isem)

    issue_in(0, nf0, sp0, isem0)
    issue_in(1, nf1, sp1, isem1)

    nh = (ntw + 1) // 2

    def pair(h, carry):
        slot(2 * h, h, nf0, sp0, out0, isem0, osem0)

        @pl.when(2 * h + 1 < ntw)
        def _():
            slot(2 * h + 1, h, nf1, sp1, out1, isem1, osem1)

        return carry

    lax.fori_loop(0, nh, pair, 0)

    pltpu.make_async_copy(out0, out_hbm.at[pl.ds(0, T), :], osem0).wait()
    pltpu.make_async_copy(out1, out_hbm.at[pl.ds(0, T), :], osem1).wait()

  return _body


def _make_kernel(n_rows):
    ntiles = n_rows // T
    assert n_rows % T == 0 and ntiles >= 2 * NW
    mesh = plsc.VectorSubcoreMesh(core_axis_name="c", subcore_axis_name="s")
    return pl.kernel(
        _make_body(ntiles),
        out_type=jax.ShapeDtypeStruct((n_rows, DIM), jnp.float32),
        mesh=mesh,
        compiler_params=pltpu.CompilerParams(needs_layout_passes=False),
        scratch_types=[
            pltpu.VMEM((NUM_TYPES * NUM_IRREPS,), jnp.float32),
            pltpu.VMEM((NUM_TYPES * NUM_SCALAR,), jnp.float32),
            pltpu.VMEM((DIM,), jnp.int32),
            pltpu.VMEM((NUM_TYPES * DIM,), jnp.float32),
            pltpu.VMEM((T, DIM), jnp.float32),
            pltpu.VMEM((T, DIM), jnp.float32),
            pltpu.VMEM((T, DIM), jnp.float32),
            pltpu.VMEM((T, DIM), jnp.float32),
            pltpu.VMEM((T,), jnp.int32),
            pltpu.VMEM((T,), jnp.int32),
            pltpu.SemaphoreType.DMA,
            pltpu.SemaphoreType.DMA,
            pltpu.SemaphoreType.DMA,
            pltpu.SemaphoreType.DMA,
        ],
    )


@jax.jit
def _run(nf, at, scales_flat, shifts_flat, sidx):
    f_a = _make_kernel(N_SPLIT)
    f_b = _make_kernel(N_ATOMS - N_SPLIT)
    out_a = f_a(nf[:N_SPLIT], at[:N_SPLIT], scales_flat, shifts_flat, sidx)
    out_b = f_b(nf[N_SPLIT:], at[N_SPLIT:], scales_flat, shifts_flat, sidx)
    return jnp.concatenate([out_a, out_b], axis=0)


def kernel(node_features, atom_types, scales, shifts):
    sidx = jnp.asarray(SCALE_IDX_NP)
    return _run(node_features, atom_types.astype(jnp.int32),
                scales.reshape(-1), shifts.reshape(-1), sidx)


# speedup vs baseline: 5.1361x; 5.1361x over previous
import jax
import jax.numpy as jnp
import numpy as np
from jax import lax
from jax.experimental import pallas as pl
from jax.experimental.pallas import tpu as pltpu
from jax.experimental.pallas import tpu_sc as plsc

N_ATOMS = 50000
NUM_TYPES = 64
_IRREPS = [(128, 1), (64, 3), (32, 5)]
NUM_SCALAR = 128
DIM = sum(m * d for m, d in _IRREPS)
NUM_IRREPS = sum(m for m, _ in _IRREPS)

_scale_idx = []
_k = 0
for _mul, _irdim in _IRREPS:
    for _ in range(_mul):
        _scale_idx += [_k] * _irdim
        _k += 1
SCALE_IDX_NP = np.asarray(_scale_idx, dtype=np.int32)
assert SCALE_IDX_NP.shape[0] == DIM

NC, NS, L = 2, 16, 16
NW = NC * NS
P = 128
QROWS = 120
NQ = DIM // QROWS
TSTRIDE = DIM + 1
HSTRIDE = NUM_SCALAR + 1
NUNITS = N_ATOMS // P
TAIL_BASE = NUNITS * P
TAIL_N = N_ATOMS - TAIL_BASE

_COLS30 = DIM // L


def _body(nf_hbm, at_hbm, scales_hbm, shifts_hbm, sidx_hbm, out_hbm,
          scales_v, shifts_v, sidx_v, table_v, htab_v,
          in0, in1, out0, out1, sp_v,
          isem0, isem1, osem0, osem1):
    wid = lax.axis_index("s") * NC + lax.axis_index("c")

    pltpu.sync_copy(scales_hbm, scales_v)
    pltpu.sync_copy(shifts_hbm, shifts_v)
    pltpu.sync_copy(sidx_hbm, sidx_v)

    iota = lax.iota(jnp.int32, L)

    def expand_row(r, carry):
        rbase = jnp.full((L,), r * NUM_IRREPS, dtype=jnp.int32)
        for cb in range(_COLS30):
            col = sidx_v[pl.ds(cb * L, L)]
            vals = plsc.load_gather(scales_v, [rbase + col])
            plsc.store_scatter(table_v, [r * TSTRIDE + cb * L + iota], vals)
        for cb in range(NUM_SCALAR // L):
            vals = shifts_v[pl.ds(r * NUM_SCALAR + cb * L, L)]
            plsc.store_scatter(htab_v, [r * HSTRIDE + cb * L + iota], vals)
        return carry

    lax.fori_loop(0, NUM_TYPES, expand_row, 0)

    ntw = (NUNITS - 1 - wid) // NW + 1

    def ubase(k):
        return (wid + k * NW) * P

    def issue_in(k, qr, in_v, isem):
        b = ubase(k)
        pltpu.async_copy(nf_hbm.at[pl.ds(qr * QROWS, QROWS), pl.ds(b, P)],
                         in_v, isem)

    def compute_q(qr, in_v, out_v):
        row0 = qr * QROWS
        row1 = row0 + QROWS
        for g in range(P // L):
            il0 = g * L
            s16 = sp_v[pl.ds(il0, L)]
            s481 = s16 * TSTRIDE
            s129 = s16 * HSTRIDE
            lo_s, hi_s = row0, min(row1, NUM_SCALAR)
            if lo_s < hi_s:
                @plsc.parallel_loop(lo_s, hi_s, 1, unroll=4)
                def _(c):
                    scc = plsc.load_gather(table_v, [s481 + c])
                    shc = plsc.load_gather(htab_v, [s129 + c])
                    nfc = in_v[c - row0, pl.ds(il0, L)]
                    out_v[c - row0, pl.ds(il0, L)] = nfc * scc + shc
            lo_n, hi_n = max(row0, NUM_SCALAR), row1
            if lo_n < hi_n:
                @plsc.parallel_loop(lo_n, hi_n, 1, unroll=4)
                def _(c):
                    scc = plsc.load_gather(table_v, [s481 + c])
                    nfc = in_v[c - row0, pl.ds(il0, L)]
                    out_v[c - row0, pl.ds(il0, L)] = nfc * scc

    def slot(k, qr, in_v, out_v, isem, osem):
        b = ubase(k)
        src = nf_hbm.at[pl.ds(qr * QROWS, QROWS), pl.ds(b, P)]
        dst = out_hbm.at[pl.ds(qr * QROWS, QROWS), pl.ds(b, P)]
        pltpu.make_async_copy(src, in_v, isem).wait()

        def drain():
            pltpu.make_async_copy(out_v, dst, osem).wait()

        if qr >= 2:
            drain()
        else:
            pl.when(k > 0)(drain)

        compute_q(qr, in_v, out_v)
        pltpu.async_copy(out_v, dst, osem)

        if qr < 2:
            issue_in(k, qr + 2, in_v, isem)
        else:
            @pl.when(k + 1 < ntw)
            def _():
                issue_in(k + 1, qr - 2, in_v, isem)

    issue_in(0, 0, in0, isem0)
    issue_in(0, 1, in1, isem1)

    def unit(k, carry):
        pltpu.sync_copy(at_hbm.at[pl.ds(ubase(k), P)], sp_v)
        slot(k, 0, in0, out0, isem0, osem0)
        slot(k, 1, in1, out1, isem1, osem1)
        slot(k, 2, in0, out0, isem0, osem0)
        slot(k, 3, in1, out1, isem1, osem1)
        return carry

    lax.fori_loop(0, ntw, unit, 0)

    pltpu.make_async_copy(out0, out_hbm.at[pl.ds(0, QROWS), pl.ds(0, P)],
                          osem0).wait()
    pltpu.make_async_copy(out1, out_hbm.at[pl.ds(0, QROWS), pl.ds(0, P)],
                          osem1).wait()


@jax.jit
def _run(nf_t, at, scales_flat, shifts_flat, sidx):
    mesh = plsc.VectorSubcoreMesh(core_axis_name="c", subcore_axis_name="s")
    f = pl.kernel(
        _body,
        out_type=jax.ShapeDtypeStruct((DIM, N_ATOMS), jnp.float32),
        mesh=mesh,
        compiler_params=pltpu.CompilerParams(needs_layout_passes=False),
        scratch_types=[
            pltpu.VMEM((NUM_TYPES * NUM_IRREPS,), jnp.float32),
            pltpu.VMEM((NUM_TYPES * NUM_SCALAR,), jnp.float32),
            pltpu.VMEM((DIM,), jnp.int32),
            pltpu.VMEM((NUM_TYPES * TSTRIDE,), jnp.float32),
            pltpu.VMEM((NUM_TYPES * HSTRIDE,), jnp.float32),
            pltpu.VMEM((QROWS, P), jnp.float32),
            pltpu.VMEM((QROWS, P), jnp.float32),
            pltpu.VMEM((QROWS, P), jnp.float32),
            pltpu.VMEM((QROWS, P), jnp.float32),
            pltpu.VMEM((P,), jnp.int32),
            pltpu.SemaphoreType.DMA,
            pltpu.SemaphoreType.DMA,
            pltpu.SemaphoreType.DMA,
            pltpu.SemaphoreType.DMA,
        ],
    )
    return f(nf_t, at, scales_flat, shifts_flat, sidx)


def kernel(node_features, atom_types, scales, shifts):
    sidx = jnp.asarray(SCALE_IDX_NP)
    nf_t = node_features.T
    at32 = atom_types.astype(jnp.int32)
    out_t = _run(nf_t, at32, scales.reshape(-1), shifts.reshape(-1), sidx)
    sp_tail = at32[TAIL_BASE:]
    scale_rows = jnp.take(scales, sp_tail, axis=0)
    scale_full = jnp.take(scale_rows, sidx, axis=1)
    shift_rows = jnp.take(shifts, sp_tail, axis=0)
    nf_tail = lax.slice(nf_t, (0, TAIL_BASE), (DIM, N_ATOMS))
    out_tail = scale_full.T * nf_tail
    out_tail = out_tail.at[:NUM_SCALAR, :].add(shift_rows.T)
    out_t = lax.dynamic_update_slice(out_t, out_tail, (0, TAIL_BASE))
    return out_t.T


# speedup vs baseline: 5.1540x; 1.0035x over previous
import jax
import jax.numpy as jnp
import numpy as np
from jax import lax
from jax.experimental import pallas as pl
from jax.experimental.pallas import tpu as pltpu
from jax.experimental.pallas import tpu_sc as plsc

N_ATOMS = 50000
NUM_TYPES = 64
_IRREPS = [(128, 1), (64, 3), (32, 5)]
NUM_SCALAR = 128
DIM = sum(m * d for m, d in _IRREPS)
NUM_IRREPS = sum(m for m, _ in _IRREPS)

_scale_idx = []
_k = 0
for _mul, _irdim in _IRREPS:
    for _ in range(_mul):
        _scale_idx += [_k] * _irdim
        _k += 1
SCALE_IDX_NP = np.asarray(_scale_idx, dtype=np.int32)
assert SCALE_IDX_NP.shape[0] == DIM

NC, NS, L = 2, 16, 16
NW = NC * NS
P = 128
QROWS = 120
NQ = DIM // QROWS
TSTRIDE = DIM + 1
HSTRIDE = NUM_SCALAR + 1
NUNITS = N_ATOMS // P
TAIL_BASE = NUNITS * P
TAIL_N = N_ATOMS - TAIL_BASE

_COLS30 = DIM // L


def _body(nf_hbm, at_hbm, scales_hbm, shifts_hbm, sidx_hbm, out_hbm,
          scales_v, shifts_v, sidx_v, table_v, htab_v,
          in0, in1, out0, out1, sp_v,
          isem0, isem1, osem0, osem1):
    wid = lax.axis_index("s") * NC + lax.axis_index("c")

    pltpu.sync_copy(scales_hbm, scales_v)
    pltpu.sync_copy(shifts_hbm, shifts_v)
    pltpu.sync_copy(sidx_hbm, sidx_v)

    iota = lax.iota(jnp.int32, L)

    def expand_row(r, carry):
        rbase = jnp.full((L,), r * NUM_IRREPS, dtype=jnp.int32)
        for cb in range(_COLS30):
            col = sidx_v[pl.ds(cb * L, L)]
            vals = plsc.load_gather(scales_v, [rbase + col])
            plsc.store_scatter(table_v, [r * TSTRIDE + cb * L + iota], vals)
        for cb in range(NUM_SCALAR // L):
            vals = shifts_v[pl.ds(r * NUM_SCALAR + cb * L, L)]
            plsc.store_scatter(htab_v, [r * HSTRIDE + cb * L + iota], vals)
        return carry

    lax.fori_loop(0, NUM_TYPES, expand_row, 0)

    ntw = (NUNITS - 1 - wid) // NW + 1

    def ubase(k):
        return (wid + k * NW) * P

    def issue_in(k, qr, in_v, isem):
        b = ubase(k)
        pltpu.async_copy(nf_hbm.at[pl.ds(qr * QROWS, QROWS), pl.ds(b, P)],
                         in_v, isem)

    def compute_q(qr, in_v, out_v):
        row0 = qr * QROWS
        row1 = row0 + QROWS
        for g in range(P // L):
            il0 = g * L
            s16 = sp_v[pl.ds(il0, L)]
            s481 = s16 * TSTRIDE
            s129 = s16 * HSTRIDE
            lo_s, hi_s = row0, min(row1, NUM_SCALAR)
            if lo_s < hi_s:
                @plsc.parallel_loop(lo_s, hi_s, 1, unroll=8)
                def _(c):
                    scc = plsc.load_gather(table_v, [s481 + c])
                    shc = plsc.load_gather(htab_v, [s129 + c])
                    nfc = in_v[c - row0, pl.ds(il0, L)]
                    out_v[c - row0, pl.ds(il0, L)] = nfc * scc + shc
            lo_n, hi_n = max(row0, NUM_SCALAR), row1
            if lo_n < hi_n:
                @plsc.parallel_loop(lo_n, hi_n, 1, unroll=8)
                def _(c):
                    scc = plsc.load_gather(table_v, [s481 + c])
                    nfc = in_v[c - row0, pl.ds(il0, L)]
                    out_v[c - row0, pl.ds(il0, L)] = nfc * scc

    def slot(k, qr, in_v, out_v, isem, osem):
        b = ubase(k)
        src = nf_hbm.at[pl.ds(qr * QROWS, QROWS), pl.ds(b, P)]
        dst = out_hbm.at[pl.ds(qr * QROWS, QROWS), pl.ds(b, P)]
        pltpu.make_async_copy(src, in_v, isem).wait()

        def drain():
            pltpu.make_async_copy(out_v, dst, osem).wait()

        if qr >= 2:
            drain()
        else:
            pl.when(k > 0)(drain)

        compute_q(qr, in_v, out_v)
        pltpu.async_copy(out_v, dst, osem)

        if qr < 2:
            issue_in(k, qr + 2, in_v, isem)
        else:
            @pl.when(k + 1 < ntw)
            def _():
                issue_in(k + 1, qr - 2, in_v, isem)

    issue_in(0, 0, in0, isem0)
    issue_in(0, 1, in1, isem1)

    def unit(k, carry):
        pltpu.sync_copy(at_hbm.at[pl.ds(ubase(k), P)], sp_v)
        slot(k, 0, in0, out0, isem0, osem0)
        slot(k, 1, in1, out1, isem1, osem1)
        slot(k, 2, in0, out0, isem0, osem0)
        slot(k, 3, in1, out1, isem1, osem1)
        return carry

    lax.fori_loop(0, ntw, unit, 0)

    pltpu.make_async_copy(out0, out_hbm.at[pl.ds(0, QROWS), pl.ds(0, P)],
                          osem0).wait()
    pltpu.make_async_copy(out1, out_hbm.at[pl.ds(0, QROWS), pl.ds(0, P)],
                          osem1).wait()


@jax.jit
def _run(nf_t, at, scales_flat, shifts_flat, sidx):
    mesh = plsc.VectorSubcoreMesh(core_axis_name="c", subcore_axis_name="s")
    f = pl.kernel(
        _body,
        out_type=jax.ShapeDtypeStruct((DIM, N_ATOMS), jnp.float32),
        mesh=mesh,
        compiler_params=pltpu.CompilerParams(needs_layout_passes=False),
        scratch_types=[
            pltpu.VMEM((NUM_TYPES * NUM_IRREPS,), jnp.float32),
            pltpu.VMEM((NUM_TYPES * NUM_SCALAR,), jnp.float32),
            pltpu.VMEM((DIM,), jnp.int32),
            pltpu.VMEM((NUM_TYPES * TSTRIDE,), jnp.float32),
            pltpu.VMEM((NUM_TYPES * HSTRIDE,), jnp.float32),
            pltpu.VMEM((QROWS, P), jnp.float32),
            pltpu.VMEM((QROWS, P), jnp.float32),
            pltpu.VMEM((QROWS, P), jnp.float32),
            pltpu.VMEM((QROWS, P), jnp.float32),
            pltpu.VMEM((P,), jnp.int32),
            pltpu.SemaphoreType.DMA,
            pltpu.SemaphoreType.DMA,
            pltpu.SemaphoreType.DMA,
            pltpu.SemaphoreType.DMA,
        ],
    )
    return f(nf_t, at, scales_flat, shifts_flat, sidx)


def kernel(node_features, atom_types, scales, shifts):
    sidx = jnp.asarray(SCALE_IDX_NP)
    nf_t = node_features.T
    at32 = atom_types.astype(jnp.int32)
    out_t = _run(nf_t, at32, scales.reshape(-1), shifts.reshape(-1), sidx)
    sp_tail = at32[TAIL_BASE:]
    scale_rows = jnp.take(scales, sp_tail, axis=0)
    scale_full = jnp.take(scale_rows, sidx, axis=1)
    shift_rows = jnp.take(shifts, sp_tail, axis=0)
    nf_tail = lax.slice(nf_t, (0, TAIL_BASE), (DIM, N_ATOMS))
    out_tail = scale_full.T * nf_tail
    out_tail = out_tail.at[:NUM_SCALAR, :].add(shift_rows.T)
    out_t = lax.dynamic_update_slice(out_t, out_tail, (0, TAIL_BASE))
    return out_t.T
